# baseline (speedup 1.0000x reference)
import functools

import jax
import jax.numpy as jnp
from jax import lax
from jax.experimental import pallas as pl
from jax.experimental.pallas import tpu as pltpu
from jax.experimental.pallas import tpu_sc as plsc

_INFO = plsc.get_sparse_core_info()
_NC = _INFO.num_cores
_NS = _INFO.num_subcores
_NW = _NC * _NS
_L = _INFO.num_lanes
_NCHUNK = 4


def _make_lookup(batch, nodes, dim):
    b_per_w = batch // _NW
    rows_per_chunk = b_per_w // _NCHUNK
    blocks_per_chunk = rows_per_chunk // _L
    mesh = plsc.VectorSubcoreMesh(core_axis_name="c", subcore_axis_name="s")

    @functools.partial(
        pl.kernel,
        mesh=mesh,
        out_type=jax.ShapeDtypeStruct((batch * dim,), jnp.float32),
        compiler_params=pltpu.CompilerParams(needs_layout_passes=False),
        scratch_types=[
            pltpu.VMEM((b_per_w,), jnp.int32),
            pltpu.VMEM((nodes * dim,), jnp.float32),
            pltpu.VMEM((b_per_w * dim,), jnp.float32),
            pltpu.SemaphoreType.DMA,
            pltpu.SemaphoreType.DMA,
        ],
    )
    def lookup(idx_hbm, table_hbm, out_hbm, idx_v, table_v, out_v, lsem, osem):
        wid = lax.axis_index("s") * _NC + lax.axis_index("c")
        base = wid * b_per_w
        ld_idx = pltpu.async_copy(idx_hbm.at[wid], idx_v, lsem)
        ld_tab = pltpu.async_copy(table_hbm, table_v, lsem)
        ld_idx.wait()
        ld_tab.wait()

        lanes = lax.iota(jnp.int32, _L)

        copies = []
        for h in range(_NCHUNK):

            @plsc.parallel_loop(h * blocks_per_chunk,
                                (h + 1) * blocks_per_chunk)
            def _block(i):
                idxf = idx_v[pl.ds(i * _L, _L)] * dim
                rowf = (i * _L + lanes) * dim
                for c in range(dim):
                    colc = lanes ^ c
                    vals = plsc.load_gather(table_v, [idxf + colc])
                    plsc.store_scatter(out_v, [rowf + colc], vals)

            off = h * rows_per_chunk * dim
            copies.append(
                pltpu.async_copy(
                    out_v.at[pl.ds(off, rows_per_chunk * dim)],
                    out_hbm.at[pl.ds(base * dim + off, rows_per_chunk * dim)],
                    osem,
                )
            )
        for cp in copies:
            cp.wait()

    return lookup


def kernel(elements, emb):
    batch = elements.shape[0]
    nodes, dim = emb.shape
    idx2d = elements.reshape(_NW, batch // _NW)
    out = _make_lookup(batch, nodes, dim)(idx2d, emb.reshape(-1))
    return out.reshape(batch, dim)


# speedup vs baseline: 1.9339x; 1.9339x over previous
import functools

import jax
import jax.numpy as jnp
from jax import lax
from jax.experimental import pallas as pl
from jax.experimental.pallas import tpu as pltpu
from jax.experimental.pallas import tpu_sc as plsc

_INFO = plsc.get_sparse_core_info()
_NC = _INFO.num_cores
_NS = _INFO.num_subcores
_NW = _NC * _NS
_CHUNK = 128


def _make_lookup(batch, nodes, dim):
    b_per_w = batch // _NW
    n_chunks = b_per_w // _CHUNK
    mesh = plsc.VectorSubcoreMesh(core_axis_name="c", subcore_axis_name="s")

    @functools.partial(
        pl.kernel,
        mesh=mesh,
        out_type=jax.ShapeDtypeStruct((batch, dim), jnp.float32),
        scratch_types=[
            pltpu.VMEM_SHARED((nodes, dim), jnp.float32),
            pltpu.VMEM((n_chunks, _CHUNK), jnp.int32),
            pltpu.VMEM((b_per_w, dim), jnp.float32),
            pltpu.SemaphoreType.DMA,
            pltpu.SemaphoreType.DMA,
        ],
    )
    def lookup(idx_hbm, table_hbm, out_hbm, table_sh, idx_v, rows_v, gsem, osem):
        cid = lax.axis_index("c")
        sid = lax.axis_index("s")
        wid = sid * _NC + cid
        base = wid * b_per_w

        @pl.when(sid == 0)
        def _stage():
            pltpu.sync_copy(table_hbm, table_sh)

        pltpu.sync_copy(idx_hbm.at[wid], idx_v)
        plsc.subcore_barrier()

        gathers = []
        for j in range(n_chunks):
            gathers.append(
                pltpu.async_copy(
                    table_sh.at[idx_v.at[j]],
                    rows_v.at[pl.ds(j * _CHUNK, _CHUNK)],
                    gsem,
                )
            )
        outs = []
        for j in range(n_chunks):
            gathers[j].wait()
            outs.append(
                pltpu.async_copy(
                    rows_v.at[pl.ds(j * _CHUNK, _CHUNK)],
                    out_hbm.at[pl.ds(base + j * _CHUNK, _CHUNK)],
                    osem,
                )
            )
        for o in outs:
            o.wait()

    return lookup


def kernel(elements, emb):
    batch = elements.shape[0]
    nodes, dim = emb.shape
    idx3d = elements.reshape(_NW, (batch // _NW) // _CHUNK, _CHUNK)
    return _make_lookup(batch, nodes, dim)(idx3d, emb)
